# Initial kernel scaffold; baseline (speedup 1.0000x reference)
import jax
import jax.numpy as jnp
from jax.experimental import pallas as pl


def kernel(x, edge_index, W1, b1, W2, b2, g1, be1, g2, be2, skip_W, lin_W, lin_b):
    raise NotImplementedError("write your pallas kernel here")


# speedup vs baseline: 18.2080x; 18.2080x over previous
import functools

import jax
import jax.numpy as jnp
from jax import lax
from jax.experimental import pallas as pl
from jax.experimental.pallas import tpu as pltpu
import jax.experimental.pallas.tpu_sc as plsc

N = 10000
E = 320000
D = 128
H = 128
O = 2
NP = 10240

NC = 2
NS = 16
NW = NC * NS
EPW = E // NW
CH = 125
NCH = EPW // CH
RPT = NP // NS

_SC_MESH = dict(core_axis_name="c", subcore_axis_name="s",
                num_cores=NC, num_subcores=NS)


def _sc_deg_body(dst_hbm, ones_hbm, zeros_hbm, out_hbm, idx_v, ones_v, acc, sem):
    c = lax.axis_index("c")
    s = lax.axis_index("s")
    pltpu.sync_copy(zeros_hbm, acc.at[pl.ds(s * RPT, RPT)])
    pltpu.sync_copy(dst_hbm.at[c, s], idx_v)
    pltpu.sync_copy(ones_hbm, ones_v)
    plsc.subcore_barrier()

    def body(j, carry):
        pltpu.sync_copy(ones_v, acc.at[idx_v.at[j]], add=True)
        return carry

    lax.fori_loop(0, NCH, body, 0)
    plsc.subcore_barrier()
    pltpu.sync_copy(acc.at[pl.ds(s * RPT, RPT)],
                    out_hbm.at[c, pl.ds(s * RPT, RPT)])


@jax.jit
def _sc_deg(dst_r, ones_hbm, zeros_hbm):
    return pl.kernel(
        _sc_deg_body,
        out_type=jax.ShapeDtypeStruct((NC, NP, H), jnp.float32),
        mesh=plsc.VectorSubcoreMesh(**_SC_MESH),
        scratch_types=[
            pltpu.VMEM((NCH, CH), jnp.int32),
            pltpu.VMEM((CH, H), jnp.float32),
            pltpu.VMEM_SHARED((NP, H), jnp.float32),
            pltpu.SemaphoreType.DMA,
        ],
    )(dst_r, ones_hbm, zeros_hbm)


def _sc_agg_body(hws_hbm, src_hbm, dst_hbm, zeros_hbm, out_hbm,
                 sidx_v, didx_v, rows_v, acc, sem):
    c = lax.axis_index("c")
    s = lax.axis_index("s")
    pltpu.sync_copy(zeros_hbm, acc.at[pl.ds(s * RPT, RPT)])
    pltpu.sync_copy(src_hbm.at[c, s], sidx_v)
    pltpu.sync_copy(dst_hbm.at[c, s], didx_v)
    plsc.subcore_barrier()

    def body(j, carry):
        pltpu.async_copy(hws_hbm.at[sidx_v.at[j]], rows_v, sem).wait()
        pltpu.sync_copy(rows_v, acc.at[didx_v.at[j]], add=True)
        return carry

    lax.fori_loop(0, NCH, body, 0)
    plsc.subcore_barrier()
    pltpu.sync_copy(acc.at[pl.ds(s * RPT, RPT)],
                    out_hbm.at[c, pl.ds(s * RPT, RPT)])


@jax.jit
def _sc_agg(hws, src_r, dst_r, zeros_hbm):
    return pl.kernel(
        _sc_agg_body,
        out_type=jax.ShapeDtypeStruct((NC, NP, H), jnp.float32),
        mesh=plsc.VectorSubcoreMesh(**_SC_MESH),
        scratch_types=[
            pltpu.VMEM((NCH, CH), jnp.int32),
            pltpu.VMEM((NCH, CH), jnp.int32),
            pltpu.VMEM((CH, H), jnp.float32),
            pltpu.VMEM_SHARED((NP, H), jnp.float32),
            pltpu.SemaphoreType.DMA,
        ],
    )(hws, src_r, dst_r, zeros_hbm)


_BR = 512


def _tc_prep_body(x_ref, p0_ref, p1_ref, w1_ref, sw_ref,
                  hws_ref, res_ref, dinv_ref):
    d = lax.rsqrt(p0_ref[:, :1] + p1_ref[:, :1] + 1.0)
    xb = x_ref[...]
    hw = jnp.dot(xb, w1_ref[...], preferred_element_type=jnp.float32)
    hws_ref[...] = hw * d
    res_ref[...] = jnp.dot(xb, sw_ref[...], preferred_element_type=jnp.float32)
    dinv_ref[...] = jnp.broadcast_to(d, hw.shape)


@jax.jit
def _tc_prep(x_pad, p0, p1, W1, skip_W):
    grid = (NP // _BR,)
    return pl.pallas_call(
        _tc_prep_body,
        grid=grid,
        in_specs=[
            pl.BlockSpec((_BR, D), lambda i: (i, 0)),
            pl.BlockSpec((_BR, H), lambda i: (i, 0)),
            pl.BlockSpec((_BR, H), lambda i: (i, 0)),
            pl.BlockSpec((D, H), lambda i: (0, 0)),
            pl.BlockSpec((D, H), lambda i: (0, 0)),
        ],
        out_specs=[
            pl.BlockSpec((_BR, H), lambda i: (i, 0)),
            pl.BlockSpec((_BR, H), lambda i: (i, 0)),
            pl.BlockSpec((_BR, H), lambda i: (i, 0)),
        ],
        out_shape=[
            jax.ShapeDtypeStruct((NP, H), jnp.float32),
            jax.ShapeDtypeStruct((NP, H), jnp.float32),
            jax.ShapeDtypeStruct((NP, H), jnp.float32),
        ],
    )(x_pad, p0, p1, W1, skip_W)


def _tc_mid_body(s0_ref, s1_ref, hws_ref, dinv_ref, b_ref, g_ref, be_ref,
                 w2_ref, out_ref):
    d = dinv_ref[:, :1]
    bns = g_ref[...] * lax.rsqrt(jnp.float32(1.0 + 1e-5))
    h = (s0_ref[...] + s1_ref[...] + hws_ref[...]) * d + b_ref[...]
    h = jnp.maximum(h * bns + be_ref[...], 0.0)
    out_ref[...] = jnp.dot(h, w2_ref[...],
                           preferred_element_type=jnp.float32) * d


@jax.jit
def _tc_mid(s0, s1, hws1, dinv, b1, g1, be1, W2):
    grid = (NP // _BR,)
    blk = pl.BlockSpec((_BR, H), lambda i: (i, 0))
    vec = pl.BlockSpec((1, H), lambda i: (0, 0))
    return pl.pallas_call(
        _tc_mid_body,
        grid=grid,
        in_specs=[blk, blk, blk, blk, vec, vec, vec,
                  pl.BlockSpec((H, H), lambda i: (0, 0))],
        out_specs=blk,
        out_shape=jax.ShapeDtypeStruct((NP, H), jnp.float32),
    )(s0, s1, hws1, dinv, b1, g1, be1, W2)


def _tc_fin_body(s0_ref, s1_ref, hws_ref, dinv_ref, res_ref, b_ref, g_ref,
                 be_ref, lw_ref, lb_ref, out_ref):
    d = dinv_ref[:, :1]
    bns = g_ref[...] * lax.rsqrt(jnp.float32(1.0 + 1e-5))
    h = (s0_ref[...] + s1_ref[...] + hws_ref[...]) * d + b_ref[...]
    h = jnp.maximum(h * bns + be_ref[...] + res_ref[...], 0.0)
    out_ref[...] = jnp.dot(h, lw_ref[...],
                           preferred_element_type=jnp.float32) + lb_ref[...]


@jax.jit
def _tc_fin(s0, s1, hws2, dinv, resid, b2, g2, be2, lin_W_pad, lin_b_pad):
    grid = (NP // _BR,)
    blk = pl.BlockSpec((_BR, H), lambda i: (i, 0))
    vec = pl.BlockSpec((1, H), lambda i: (0, 0))
    return pl.pallas_call(
        _tc_fin_body,
        grid=grid,
        in_specs=[blk, blk, blk, blk, blk, vec, vec, vec,
                  pl.BlockSpec((H, H), lambda i: (0, 0)), vec],
        out_specs=blk,
        out_shape=jax.ShapeDtypeStruct((NP, H), jnp.float32),
    )(s0, s1, hws2, dinv, resid, b2, g2, be2, lin_W_pad, lin_b_pad)


def kernel(x, edge_index, W1, b1, W2, b2, g1, be1, g2, be2,
           skip_W, lin_W, lin_b):
    src_r = edge_index[0].reshape(NC, NS, NCH, CH)
    dst_r = edge_index[1].reshape(NC, NS, NCH, CH)
    x_pad = jnp.pad(x, ((0, NP - N), (0, 0)))

    zeros_hbm = jnp.zeros((RPT, H), jnp.float32)
    ones_hbm = jnp.ones((CH, H), jnp.float32)

    degp = _sc_deg(dst_r, ones_hbm, zeros_hbm)
    hws1, resid, dinv = _tc_prep(x_pad, degp[0], degp[1], W1, skip_W)

    agg1 = _sc_agg(hws1, src_r, dst_r, zeros_hbm)
    hws2 = _tc_mid(agg1[0], agg1[1], hws1, dinv,
                   b1.reshape(1, H), g1.reshape(1, H), be1.reshape(1, H), W2)

    agg2 = _sc_agg(hws2, src_r, dst_r, zeros_hbm)
    lin_W_pad = jnp.pad(lin_W, ((0, 0), (0, H - O)))
    lin_b_pad = jnp.pad(lin_b, ((0, H - O),)).reshape(1, H)
    out = _tc_fin(agg2[0], agg2[1], hws2, dinv, resid,
                  b2.reshape(1, H), g2.reshape(1, H), be2.reshape(1, H),
                  lin_W_pad, lin_b_pad)
    return out[:N, :O]


# speedup vs baseline: 22.9105x; 1.2583x over previous
import functools

import jax
import jax.numpy as jnp
from jax import lax
from jax.experimental import pallas as pl
from jax.experimental.pallas import tpu as pltpu
import jax.experimental.pallas.tpu_sc as plsc

N = 10000
E = 320000
D = 128
H = 128
O = 2
NP = 10240

NC = 2
NS = 16
NW = NC * NS
EPW = E // NW
CH = 80
NCH = EPW // CH
RPT = NP // NS

_SC_MESH = dict(core_axis_name="c", subcore_axis_name="s",
                num_cores=NC, num_subcores=NS)


def _sc_deg_body(dst_hbm, ones_hbm, zeros_hbm, out_hbm, idx_v, ones_v, acc, sem):
    c = lax.axis_index("c")
    s = lax.axis_index("s")
    pltpu.sync_copy(zeros_hbm, acc.at[pl.ds(s * RPT, RPT)])
    pltpu.sync_copy(dst_hbm.at[c, s], idx_v)
    pltpu.sync_copy(ones_hbm, ones_v)
    plsc.subcore_barrier()

    def body(j, carry):
        pltpu.sync_copy(ones_v, acc.at[idx_v.at[j]], add=True)
        return carry

    lax.fori_loop(0, NCH, body, 0)
    plsc.subcore_barrier()
    pltpu.sync_copy(acc.at[pl.ds(s * RPT, RPT)],
                    out_hbm.at[c, pl.ds(s * RPT, RPT)])


@jax.jit
def _sc_deg(dst_r, ones_hbm, zeros_hbm):
    return pl.kernel(
        _sc_deg_body,
        out_type=jax.ShapeDtypeStruct((NC, NP, H), jnp.float32),
        mesh=plsc.VectorSubcoreMesh(**_SC_MESH),
        scratch_types=[
            pltpu.VMEM((NCH, CH), jnp.int32),
            pltpu.VMEM((CH, H), jnp.float32),
            pltpu.VMEM_SHARED((NP, H), jnp.float32),
            pltpu.SemaphoreType.DMA,
        ],
    )(dst_r, ones_hbm, zeros_hbm)


def _sc_agg_body(hws_hbm, src_hbm, dst_hbm, zeros_hbm, out_hbm,
                 sidx_v, didx_v, rows0, rows1, acc,
                 sem0, sem1, semd0, semd1):
    c = lax.axis_index("c")
    s = lax.axis_index("s")
    pltpu.sync_copy(zeros_hbm, acc.at[pl.ds(s * RPT, RPT)])
    pltpu.sync_copy(src_hbm.at[c, s], sidx_v)
    plsc.subcore_barrier()

    rows = (rows0, rows1)
    sems = (sem0, sem1)
    semd = (semd0, semd1)
    for b in range(2):
        pltpu.async_copy(dst_hbm.at[c, s, b], didx_v.at[pl.ds(b, 1)], semd[b])
        pltpu.async_copy(hws_hbm.at[sidx_v.at[b]], rows[b], sems[b])

    @pl.loop(0, NCH, step=2)
    def _chunks(j):
        for b in range(2):
            jj = j + b
            pltpu.make_async_copy(hws_hbm.at[sidx_v.at[jj]],
                                  rows[b], sems[b]).wait()
            pltpu.make_async_copy(dst_hbm.at[c, s, jj],
                                  didx_v.at[pl.ds(b, 1)], semd[b]).wait()
            pltpu.sync_copy(rows[b], acc.at[didx_v.at[b]], add=True)
            nxt = jnp.minimum(jj + 2, NCH - 1)
            pltpu.async_copy(dst_hbm.at[c, s, nxt],
                             didx_v.at[pl.ds(b, 1)], semd[b])
            pltpu.async_copy(hws_hbm.at[sidx_v.at[nxt]], rows[b], sems[b])

    for b in range(2):
        pltpu.make_async_copy(hws_hbm.at[sidx_v.at[NCH - 1]],
                              rows[b], sems[b]).wait()
        pltpu.make_async_copy(dst_hbm.at[c, s, NCH - 1],
                              didx_v.at[pl.ds(b, 1)], semd[b]).wait()

    plsc.subcore_barrier()
    pltpu.sync_copy(acc.at[pl.ds(s * RPT, RPT)],
                    out_hbm.at[c, pl.ds(s * RPT, RPT)])


@jax.jit
def _sc_agg(hws, src_r, dst_r, zeros_hbm):
    return pl.kernel(
        _sc_agg_body,
        out_type=jax.ShapeDtypeStruct((NC, NP, H), jnp.float32),
        mesh=plsc.VectorSubcoreMesh(**_SC_MESH),
        scratch_types=[
            pltpu.VMEM((NCH, CH), jnp.int32),
            pltpu.VMEM((2, CH), jnp.int32),
            pltpu.VMEM((CH, H), jnp.float32),
            pltpu.VMEM((CH, H), jnp.float32),
            pltpu.VMEM_SHARED((NP, H), jnp.float32),
            pltpu.SemaphoreType.DMA,
            pltpu.SemaphoreType.DMA,
            pltpu.SemaphoreType.DMA,
            pltpu.SemaphoreType.DMA,
        ],
    )(hws, src_r, dst_r, zeros_hbm)


_BR = 512


def _tc_prep_body(x_ref, p0_ref, p1_ref, w1_ref, sw_ref,
                  hws_ref, res_ref, dinv_ref):
    d = lax.rsqrt(p0_ref[:, :1] + p1_ref[:, :1] + 1.0)
    xb = x_ref[...]
    hw = jnp.dot(xb, w1_ref[...], preferred_element_type=jnp.float32)
    hws_ref[...] = hw * d
    res_ref[...] = jnp.dot(xb, sw_ref[...], preferred_element_type=jnp.float32)
    dinv_ref[...] = jnp.broadcast_to(d, hw.shape)


@jax.jit
def _tc_prep(x_pad, p0, p1, W1, skip_W):
    grid = (NP // _BR,)
    return pl.pallas_call(
        _tc_prep_body,
        grid=grid,
        in_specs=[
            pl.BlockSpec((_BR, D), lambda i: (i, 0)),
            pl.BlockSpec((_BR, H), lambda i: (i, 0)),
            pl.BlockSpec((_BR, H), lambda i: (i, 0)),
            pl.BlockSpec((D, H), lambda i: (0, 0)),
            pl.BlockSpec((D, H), lambda i: (0, 0)),
        ],
        out_specs=[
            pl.BlockSpec((_BR, H), lambda i: (i, 0)),
            pl.BlockSpec((_BR, H), lambda i: (i, 0)),
            pl.BlockSpec((_BR, H), lambda i: (i, 0)),
        ],
        out_shape=[
            jax.ShapeDtypeStruct((NP, H), jnp.float32),
            jax.ShapeDtypeStruct((NP, H), jnp.float32),
            jax.ShapeDtypeStruct((NP, H), jnp.float32),
        ],
    )(x_pad, p0, p1, W1, skip_W)


def _tc_mid_body(s0_ref, s1_ref, hws_ref, dinv_ref, b_ref, g_ref, be_ref,
                 w2_ref, out_ref):
    d = dinv_ref[:, :1]
    bns = g_ref[...] * lax.rsqrt(jnp.float32(1.0 + 1e-5))
    h = (s0_ref[...] + s1_ref[...] + hws_ref[...]) * d + b_ref[...]
    h = jnp.maximum(h * bns + be_ref[...], 0.0)
    out_ref[...] = jnp.dot(h, w2_ref[...],
                           preferred_element_type=jnp.float32) * d


@jax.jit
def _tc_mid(s0, s1, hws1, dinv, b1, g1, be1, W2):
    grid = (NP // _BR,)
    blk = pl.BlockSpec((_BR, H), lambda i: (i, 0))
    vec = pl.BlockSpec((1, H), lambda i: (0, 0))
    return pl.pallas_call(
        _tc_mid_body,
        grid=grid,
        in_specs=[blk, blk, blk, blk, vec, vec, vec,
                  pl.BlockSpec((H, H), lambda i: (0, 0))],
        out_specs=blk,
        out_shape=jax.ShapeDtypeStruct((NP, H), jnp.float32),
    )(s0, s1, hws1, dinv, b1, g1, be1, W2)


def _tc_fin_body(s0_ref, s1_ref, hws_ref, dinv_ref, res_ref, b_ref, g_ref,
                 be_ref, lw_ref, lb_ref, out_ref):
    d = dinv_ref[:, :1]
    bns = g_ref[...] * lax.rsqrt(jnp.float32(1.0 + 1e-5))
    h = (s0_ref[...] + s1_ref[...] + hws_ref[...]) * d + b_ref[...]
    h = jnp.maximum(h * bns + be_ref[...] + res_ref[...], 0.0)
    out_ref[...] = jnp.dot(h, lw_ref[...],
                           preferred_element_type=jnp.float32) + lb_ref[...]


@jax.jit
def _tc_fin(s0, s1, hws2, dinv, resid, b2, g2, be2, lin_W_pad, lin_b_pad):
    grid = (NP // _BR,)
    blk = pl.BlockSpec((_BR, H), lambda i: (i, 0))
    vec = pl.BlockSpec((1, H), lambda i: (0, 0))
    return pl.pallas_call(
        _tc_fin_body,
        grid=grid,
        in_specs=[blk, blk, blk, blk, blk, vec, vec, vec,
                  pl.BlockSpec((H, H), lambda i: (0, 0)), vec],
        out_specs=blk,
        out_shape=jax.ShapeDtypeStruct((NP, H), jnp.float32),
    )(s0, s1, hws2, dinv, resid, b2, g2, be2, lin_W_pad, lin_b_pad)


def kernel(x, edge_index, W1, b1, W2, b2, g1, be1, g2, be2,
           skip_W, lin_W, lin_b):
    src_r = edge_index[0].reshape(NC, NS, NCH, CH)
    dst_r = edge_index[1].reshape(NC, NS, NCH, CH)
    dst_r5 = edge_index[1].reshape(NC, NS, NCH, 1, CH)
    x_pad = jnp.pad(x, ((0, NP - N), (0, 0)))

    zeros_hbm = jnp.zeros((RPT, H), jnp.float32)
    ones_hbm = jnp.ones((CH, H), jnp.float32)

    degp = _sc_deg(dst_r, ones_hbm, zeros_hbm)
    hws1, resid, dinv = _tc_prep(x_pad, degp[0], degp[1], W1, skip_W)

    agg1 = _sc_agg(hws1, src_r, dst_r5, zeros_hbm)
    hws2 = _tc_mid(agg1[0], agg1[1], hws1, dinv,
                   b1.reshape(1, H), g1.reshape(1, H), be1.reshape(1, H), W2)

    agg2 = _sc_agg(hws2, src_r, dst_r5, zeros_hbm)
    lin_W_pad = jnp.pad(lin_W, ((0, 0), (0, H - O)))
    lin_b_pad = jnp.pad(lin_b, ((0, H - O),)).reshape(1, H)
    out = _tc_fin(agg2[0], agg2[1], hws2, dinv, resid,
                  b2.reshape(1, H), g2.reshape(1, H), be2.reshape(1, H),
                  lin_W_pad, lin_b_pad)
    return out[:N, :O]


# speedup vs baseline: 24.2125x; 1.0568x over previous
import functools

import jax
import jax.numpy as jnp
from jax import lax
from jax.experimental import pallas as pl
from jax.experimental.pallas import tpu as pltpu
import jax.experimental.pallas.tpu_sc as plsc

N = 10000
E = 320000
D = 128
H = 128
O = 2
NP = 10240

NC = 2
NS = 16
NW = NC * NS
EPW = E // NW
CH = 125
NCH = EPW // CH
RPT = NP // NS

_SC_MESH = dict(core_axis_name="c", subcore_axis_name="s",
                num_cores=NC, num_subcores=NS)


def _sc_deg_body(dst_hbm, ones_hbm, zeros_hbm, out_hbm, idx_v, ones_v, acc, sem):
    c = lax.axis_index("c")
    s = lax.axis_index("s")
    pltpu.sync_copy(zeros_hbm, acc.at[pl.ds(s * RPT, RPT)])
    pltpu.sync_copy(dst_hbm.at[c, s], idx_v)
    pltpu.sync_copy(ones_hbm, ones_v)
    plsc.subcore_barrier()

    def body(j, carry):
        pltpu.sync_copy(ones_v, acc.at[idx_v.at[j]], add=True)
        return carry

    lax.fori_loop(0, NCH, body, 0)
    plsc.subcore_barrier()
    pltpu.sync_copy(acc.at[pl.ds(s * RPT, RPT)],
                    out_hbm.at[c, pl.ds(s * RPT, RPT)])


@jax.jit
def _sc_deg(dst_r, ones_hbm, zeros_hbm):
    return pl.kernel(
        _sc_deg_body,
        out_type=jax.ShapeDtypeStruct((NC, NP, H), jnp.float32),
        mesh=plsc.VectorSubcoreMesh(**_SC_MESH),
        scratch_types=[
            pltpu.VMEM((NCH, CH), jnp.int32),
            pltpu.VMEM((CH, H), jnp.float32),
            pltpu.VMEM_SHARED((NP, H), jnp.float32),
            pltpu.SemaphoreType.DMA,
        ],
    )(dst_r, ones_hbm, zeros_hbm)


def _sc_agg_body(hws_hbm, src_hbm, dst_hbm, zeros_hbm, out_hbm,
                 sidx_v, didx_v, rows0, rows1, acc,
                 sem0, sem1, semd0, semd1):
    c = lax.axis_index("c")
    s = lax.axis_index("s")
    pltpu.sync_copy(zeros_hbm, acc.at[pl.ds(s * RPT, RPT)])
    pltpu.sync_copy(src_hbm.at[c, s], sidx_v)
    plsc.subcore_barrier()

    rows = (rows0, rows1)
    sems = (sem0, sem1)
    semd = (semd0, semd1)
    for b in range(2):
        pltpu.async_copy(dst_hbm.at[c, s, b], didx_v.at[pl.ds(b, 1)], semd[b])
        pltpu.async_copy(hws_hbm.at[sidx_v.at[b]], rows[b], sems[b])

    @pl.loop(0, NCH, step=2)
    def _chunks(j):
        for b in range(2):
            jj = j + b
            pltpu.make_async_copy(hws_hbm.at[sidx_v.at[jj]],
                                  rows[b], sems[b]).wait()
            pltpu.make_async_copy(dst_hbm.at[c, s, jj],
                                  didx_v.at[pl.ds(b, 1)], semd[b]).wait()
            pltpu.sync_copy(rows[b], acc.at[didx_v.at[b]], add=True)
            nxt = jnp.minimum(jj + 2, NCH - 1)
            pltpu.async_copy(dst_hbm.at[c, s, nxt],
                             didx_v.at[pl.ds(b, 1)], semd[b])
            pltpu.async_copy(hws_hbm.at[sidx_v.at[nxt]], rows[b], sems[b])

    for b in range(2):
        pltpu.make_async_copy(hws_hbm.at[sidx_v.at[NCH - 1]],
                              rows[b], sems[b]).wait()
        pltpu.make_async_copy(dst_hbm.at[c, s, NCH - 1],
                              didx_v.at[pl.ds(b, 1)], semd[b]).wait()

    plsc.subcore_barrier()
    pltpu.sync_copy(acc.at[pl.ds(s * RPT, RPT)],
                    out_hbm.at[c, pl.ds(s * RPT, RPT)])


@jax.jit
def _sc_agg(hws, src_r, dst_r, zeros_hbm):
    return pl.kernel(
        _sc_agg_body,
        out_type=jax.ShapeDtypeStruct((NC, NP, H), jnp.float32),
        mesh=plsc.VectorSubcoreMesh(**_SC_MESH),
        scratch_types=[
            pltpu.VMEM((NCH, CH), jnp.int32),
            pltpu.VMEM((2, CH), jnp.int32),
            pltpu.VMEM((CH, H), jnp.float32),
            pltpu.VMEM((CH, H), jnp.float32),
            pltpu.VMEM_SHARED((NP, H), jnp.float32),
            pltpu.SemaphoreType.DMA,
            pltpu.SemaphoreType.DMA,
            pltpu.SemaphoreType.DMA,
            pltpu.SemaphoreType.DMA,
        ],
    )(hws, src_r, dst_r, zeros_hbm)


_BR = 512


def _tc_prep_body(x_ref, p0_ref, p1_ref, w1_ref, sw_ref,
                  hws_ref, res_ref, dinv_ref):
    d = lax.rsqrt(p0_ref[:, :1] + p1_ref[:, :1] + 1.0)
    xb = x_ref[...]
    hw = jnp.dot(xb, w1_ref[...], preferred_element_type=jnp.float32)
    hws_ref[...] = hw * d
    res_ref[...] = jnp.dot(xb, sw_ref[...], preferred_element_type=jnp.float32)
    dinv_ref[...] = jnp.broadcast_to(d, hw.shape)


@jax.jit
def _tc_prep(x_pad, p0, p1, W1, skip_W):
    grid = (NP // _BR,)
    return pl.pallas_call(
        _tc_prep_body,
        grid=grid,
        in_specs=[
            pl.BlockSpec((_BR, D), lambda i: (i, 0)),
            pl.BlockSpec((_BR, H), lambda i: (i, 0)),
            pl.BlockSpec((_BR, H), lambda i: (i, 0)),
            pl.BlockSpec((D, H), lambda i: (0, 0)),
            pl.BlockSpec((D, H), lambda i: (0, 0)),
        ],
        out_specs=[
            pl.BlockSpec((_BR, H), lambda i: (i, 0)),
            pl.BlockSpec((_BR, H), lambda i: (i, 0)),
            pl.BlockSpec((_BR, H), lambda i: (i, 0)),
        ],
        out_shape=[
            jax.ShapeDtypeStruct((NP, H), jnp.float32),
            jax.ShapeDtypeStruct((NP, H), jnp.float32),
            jax.ShapeDtypeStruct((NP, H), jnp.float32),
        ],
    )(x_pad, p0, p1, W1, skip_W)


def _tc_mid_body(s0_ref, s1_ref, hws_ref, dinv_ref, b_ref, g_ref, be_ref,
                 w2_ref, out_ref):
    d = dinv_ref[:, :1]
    bns = g_ref[...] * lax.rsqrt(jnp.float32(1.0 + 1e-5))
    h = (s0_ref[...] + s1_ref[...] + hws_ref[...]) * d + b_ref[...]
    h = jnp.maximum(h * bns + be_ref[...], 0.0)
    out_ref[...] = jnp.dot(h, w2_ref[...],
                           preferred_element_type=jnp.float32) * d


@jax.jit
def _tc_mid(s0, s1, hws1, dinv, b1, g1, be1, W2):
    grid = (NP // _BR,)
    blk = pl.BlockSpec((_BR, H), lambda i: (i, 0))
    vec = pl.BlockSpec((1, H), lambda i: (0, 0))
    return pl.pallas_call(
        _tc_mid_body,
        grid=grid,
        in_specs=[blk, blk, blk, blk, vec, vec, vec,
                  pl.BlockSpec((H, H), lambda i: (0, 0))],
        out_specs=blk,
        out_shape=jax.ShapeDtypeStruct((NP, H), jnp.float32),
    )(s0, s1, hws1, dinv, b1, g1, be1, W2)


def _tc_fin_body(s0_ref, s1_ref, hws_ref, dinv_ref, res_ref, b_ref, g_ref,
                 be_ref, lw_ref, lb_ref, out_ref):
    d = dinv_ref[:, :1]
    bns = g_ref[...] * lax.rsqrt(jnp.float32(1.0 + 1e-5))
    h = (s0_ref[...] + s1_ref[...] + hws_ref[...]) * d + b_ref[...]
    h = jnp.maximum(h * bns + be_ref[...] + res_ref[...], 0.0)
    out_ref[...] = jnp.dot(h, lw_ref[...],
                           preferred_element_type=jnp.float32) + lb_ref[...]


@jax.jit
def _tc_fin(s0, s1, hws2, dinv, resid, b2, g2, be2, lin_W_pad, lin_b_pad):
    grid = (NP // _BR,)
    blk = pl.BlockSpec((_BR, H), lambda i: (i, 0))
    vec = pl.BlockSpec((1, H), lambda i: (0, 0))
    return pl.pallas_call(
        _tc_fin_body,
        grid=grid,
        in_specs=[blk, blk, blk, blk, blk, vec, vec, vec,
                  pl.BlockSpec((H, H), lambda i: (0, 0)), vec],
        out_specs=blk,
        out_shape=jax.ShapeDtypeStruct((NP, H), jnp.float32),
    )(s0, s1, hws2, dinv, resid, b2, g2, be2, lin_W_pad, lin_b_pad)


def kernel(x, edge_index, W1, b1, W2, b2, g1, be1, g2, be2,
           skip_W, lin_W, lin_b):
    src_r = edge_index[0].reshape(NC, NS, NCH, CH)
    dst_r = edge_index[1].reshape(NC, NS, NCH, CH)
    dst_r5 = edge_index[1].reshape(NC, NS, NCH, 1, CH)
    x_pad = jnp.pad(x, ((0, NP - N), (0, 0)))

    zeros_hbm = jnp.zeros((RPT, H), jnp.float32)
    ones_hbm = jnp.ones((CH, H), jnp.float32)

    degp = _sc_deg(dst_r, ones_hbm, zeros_hbm)
    hws1, resid, dinv = _tc_prep(x_pad, degp[0], degp[1], W1, skip_W)

    agg1 = _sc_agg(hws1, src_r, dst_r5, zeros_hbm)
    hws2 = _tc_mid(agg1[0], agg1[1], hws1, dinv,
                   b1.reshape(1, H), g1.reshape(1, H), be1.reshape(1, H), W2)

    agg2 = _sc_agg(hws2, src_r, dst_r5, zeros_hbm)
    lin_W_pad = jnp.pad(lin_W, ((0, 0), (0, H - O)))
    lin_b_pad = jnp.pad(lin_b, ((0, H - O),)).reshape(1, H)
    out = _tc_fin(agg2[0], agg2[1], hws2, dinv, resid,
                  b2.reshape(1, H), g2.reshape(1, H), be2.reshape(1, H),
                  lin_W_pad, lin_b_pad)
    return out[:N, :O]


# speedup vs baseline: 24.3760x; 1.0068x over previous
import functools

import jax
import jax.numpy as jnp
from jax import lax
from jax.experimental import pallas as pl
from jax.experimental.pallas import tpu as pltpu
import jax.experimental.pallas.tpu_sc as plsc

N = 10000
E = 320000
D = 128
H = 128
O = 2
NP = 10240

NC = 2
NS = 16
NW = NC * NS
EPW = E // NW
CH = 125
NCH = EPW // CH
RPT = NP // NS

_SC_MESH = dict(core_axis_name="c", subcore_axis_name="s",
                num_cores=NC, num_subcores=NS)


def _sc_deg_body(dst_hbm, ones_hbm, zeros_hbm, out_hbm, idx_v, ones_v, acc, sem):
    c = lax.axis_index("c")
    s = lax.axis_index("s")
    pltpu.sync_copy(zeros_hbm, acc.at[pl.ds(s * RPT, RPT)])
    pltpu.sync_copy(dst_hbm.at[c, s], idx_v)
    pltpu.sync_copy(ones_hbm, ones_v)
    plsc.subcore_barrier()

    def body(j, carry):
        pltpu.sync_copy(ones_v, acc.at[idx_v.at[j]], add=True)
        return carry

    lax.fori_loop(0, NCH, body, 0)
    plsc.subcore_barrier()
    pltpu.sync_copy(acc.at[pl.ds(s * RPT, RPT)],
                    out_hbm.at[c, pl.ds(s * RPT, RPT)])


@jax.jit
def _sc_deg(dst_r, ones_hbm, zeros_hbm):
    return pl.kernel(
        _sc_deg_body,
        out_type=jax.ShapeDtypeStruct((NC, NP, H), jnp.float32),
        mesh=plsc.VectorSubcoreMesh(**_SC_MESH),
        scratch_types=[
            pltpu.VMEM((NCH, CH), jnp.int32),
            pltpu.VMEM((CH, H), jnp.float32),
            pltpu.VMEM_SHARED((NP, H), jnp.float32),
            pltpu.SemaphoreType.DMA,
        ],
    )(dst_r, ones_hbm, zeros_hbm)


def _sc_agg_body(hws_hbm, src_hbm, dst_hbm, zeros_hbm, out_hbm,
                 sidx_v, didx_v, rows0, rows1, acc,
                 sem0, sem1, semd0, semd1):
    c = lax.axis_index("c")
    s = lax.axis_index("s")
    pltpu.sync_copy(zeros_hbm, acc.at[pl.ds(s * RPT, RPT)])
    pltpu.sync_copy(src_hbm.at[c, s], sidx_v)
    plsc.subcore_barrier()

    rows = (rows0, rows1)
    sems = (sem0, sem1)
    semd = (semd0, semd1)
    for b in range(2):
        pltpu.async_copy(dst_hbm.at[c, s, b], didx_v.at[pl.ds(b, 1)], semd[b])
        pltpu.async_copy(hws_hbm.at[sidx_v.at[b]], rows[b], sems[b])

    @pl.loop(0, NCH, step=2)
    def _chunks(j):
        for b in range(2):
            jj = j + b
            pltpu.make_async_copy(hws_hbm.at[sidx_v.at[jj]],
                                  rows[b], sems[b]).wait()
            pltpu.make_async_copy(dst_hbm.at[c, s, jj],
                                  didx_v.at[pl.ds(b, 1)], semd[b]).wait()
            pltpu.sync_copy(rows[b], acc.at[didx_v.at[b]], add=True)
            nxt = jnp.minimum(jj + 2, NCH - 1)
            pltpu.async_copy(dst_hbm.at[c, s, nxt],
                             didx_v.at[pl.ds(b, 1)], semd[b])
            pltpu.async_copy(hws_hbm.at[sidx_v.at[nxt]], rows[b], sems[b])

    for b in range(2):
        pltpu.make_async_copy(hws_hbm.at[sidx_v.at[NCH - 1]],
                              rows[b], sems[b]).wait()
        pltpu.make_async_copy(dst_hbm.at[c, s, NCH - 1],
                              didx_v.at[pl.ds(b, 1)], semd[b]).wait()

    plsc.subcore_barrier()
    pltpu.sync_copy(acc.at[pl.ds(s * RPT, RPT)],
                    out_hbm.at[c, pl.ds(s * RPT, RPT)])


@jax.jit
def _sc_agg(hws, src_r, dst_r, zeros_hbm):
    return pl.kernel(
        _sc_agg_body,
        out_type=jax.ShapeDtypeStruct((NC, NP, H), jnp.float32),
        mesh=plsc.VectorSubcoreMesh(**_SC_MESH),
        scratch_types=[
            pltpu.VMEM((NCH, CH), jnp.int32),
            pltpu.VMEM((2, CH), jnp.int32),
            pltpu.VMEM((CH, H), jnp.float32),
            pltpu.VMEM((CH, H), jnp.float32),
            pltpu.VMEM_SHARED((NP, H), jnp.float32),
            pltpu.SemaphoreType.DMA,
            pltpu.SemaphoreType.DMA,
            pltpu.SemaphoreType.DMA,
            pltpu.SemaphoreType.DMA,
        ],
    )(hws, src_r, dst_r, zeros_hbm)


_BR = 512


def _tc_mm_body(x_ref, w1_ref, sw_ref, hw_ref, res_ref):
    xb = x_ref[...]
    hw_ref[...] = jnp.dot(xb, w1_ref[...], preferred_element_type=jnp.float32)
    res_ref[...] = jnp.dot(xb, sw_ref[...], preferred_element_type=jnp.float32)


@jax.jit
def _tc_mm(x_pad, W1, skip_W):
    grid = (NP // _BR,)
    return pl.pallas_call(
        _tc_mm_body,
        grid=grid,
        in_specs=[
            pl.BlockSpec((_BR, D), lambda i: (i, 0)),
            pl.BlockSpec((D, H), lambda i: (0, 0)),
            pl.BlockSpec((D, H), lambda i: (0, 0)),
        ],
        out_specs=[
            pl.BlockSpec((_BR, H), lambda i: (i, 0)),
            pl.BlockSpec((_BR, H), lambda i: (i, 0)),
        ],
        out_shape=[
            jax.ShapeDtypeStruct((NP, H), jnp.float32),
            jax.ShapeDtypeStruct((NP, H), jnp.float32),
        ],
    )(x_pad, W1, skip_W)


def _tc_scale_body(hw_ref, p0_ref, p1_ref, hws_ref, dinv_ref):
    d = lax.rsqrt(p0_ref[:, :1] + p1_ref[:, :1] + 1.0)
    hw = hw_ref[...]
    hws_ref[...] = hw * d
    dinv_ref[...] = jnp.broadcast_to(d, hw.shape)


@jax.jit
def _tc_scale(hw1, p0, p1):
    grid = (NP // _BR,)
    blk = pl.BlockSpec((_BR, H), lambda i: (i, 0))
    return pl.pallas_call(
        _tc_scale_body,
        grid=grid,
        in_specs=[blk, blk, blk],
        out_specs=[blk, blk],
        out_shape=[
            jax.ShapeDtypeStruct((NP, H), jnp.float32),
            jax.ShapeDtypeStruct((NP, H), jnp.float32),
        ],
    )(hw1, p0, p1)


def _tc_mid_body(s0_ref, s1_ref, hws_ref, dinv_ref, b_ref, g_ref, be_ref,
                 w2_ref, out_ref):
    d = dinv_ref[:, :1]
    bns = g_ref[...] * lax.rsqrt(jnp.float32(1.0 + 1e-5))
    h = (s0_ref[...] + s1_ref[...] + hws_ref[...]) * d + b_ref[...]
    h = jnp.maximum(h * bns + be_ref[...], 0.0)
    out_ref[...] = jnp.dot(h, w2_ref[...],
                           preferred_element_type=jnp.float32) * d


@jax.jit
def _tc_mid(s0, s1, hws1, dinv, b1, g1, be1, W2):
    grid = (NP // _BR,)
    blk = pl.BlockSpec((_BR, H), lambda i: (i, 0))
    vec = pl.BlockSpec((1, H), lambda i: (0, 0))
    return pl.pallas_call(
        _tc_mid_body,
        grid=grid,
        in_specs=[blk, blk, blk, blk, vec, vec, vec,
                  pl.BlockSpec((H, H), lambda i: (0, 0))],
        out_specs=blk,
        out_shape=jax.ShapeDtypeStruct((NP, H), jnp.float32),
    )(s0, s1, hws1, dinv, b1, g1, be1, W2)


def _tc_fin_body(s0_ref, s1_ref, hws_ref, dinv_ref, res_ref, b_ref, g_ref,
                 be_ref, lw_ref, lb_ref, out_ref):
    d = dinv_ref[:, :1]
    bns = g_ref[...] * lax.rsqrt(jnp.float32(1.0 + 1e-5))
    h = (s0_ref[...] + s1_ref[...] + hws_ref[...]) * d + b_ref[...]
    h = jnp.maximum(h * bns + be_ref[...] + res_ref[...], 0.0)
    out_ref[...] = jnp.dot(h, lw_ref[...],
                           preferred_element_type=jnp.float32) + lb_ref[...]


@jax.jit
def _tc_fin(s0, s1, hws2, dinv, resid, b2, g2, be2, lin_W_pad, lin_b_pad):
    grid = (NP // _BR,)
    blk = pl.BlockSpec((_BR, H), lambda i: (i, 0))
    vec = pl.BlockSpec((1, H), lambda i: (0, 0))
    return pl.pallas_call(
        _tc_fin_body,
        grid=grid,
        in_specs=[blk, blk, blk, blk, blk, vec, vec, vec,
                  pl.BlockSpec((H, H), lambda i: (0, 0)), vec],
        out_specs=blk,
        out_shape=jax.ShapeDtypeStruct((NP, H), jnp.float32),
    )(s0, s1, hws2, dinv, resid, b2, g2, be2, lin_W_pad, lin_b_pad)


def kernel(x, edge_index, W1, b1, W2, b2, g1, be1, g2, be2,
           skip_W, lin_W, lin_b):
    src_r = edge_index[0].reshape(NC, NS, NCH, CH)
    dst_r = edge_index[1].reshape(NC, NS, NCH, CH)
    dst_r5 = edge_index[1].reshape(NC, NS, NCH, 1, CH)
    x_pad = jnp.pad(x, ((0, NP - N), (0, 0)))

    zeros_hbm = jnp.zeros((RPT, H), jnp.float32)
    ones_hbm = jnp.ones((CH, H), jnp.float32)

    degp = _sc_deg(dst_r, ones_hbm, zeros_hbm)
    hw1, resid = _tc_mm(x_pad, W1, skip_W)
    hws1, dinv = _tc_scale(hw1, degp[0], degp[1])

    agg1 = _sc_agg(hws1, src_r, dst_r5, zeros_hbm)
    hws2 = _tc_mid(agg1[0], agg1[1], hws1, dinv,
                   b1.reshape(1, H), g1.reshape(1, H), be1.reshape(1, H), W2)

    agg2 = _sc_agg(hws2, src_r, dst_r5, zeros_hbm)
    lin_W_pad = jnp.pad(lin_W, ((0, 0), (0, H - O)))
    lin_b_pad = jnp.pad(lin_b, ((0, H - O),)).reshape(1, H)
    out = _tc_fin(agg2[0], agg2[1], hws2, dinv, resid,
                  b2.reshape(1, H), g2.reshape(1, H), be2.reshape(1, H),
                  lin_W_pad, lin_b_pad)
    return out[:N, :O]


# speedup vs baseline: 28.1030x; 1.1529x over previous
import functools

import jax
import jax.numpy as jnp
from jax import lax
from jax.experimental import pallas as pl
from jax.experimental.pallas import tpu as pltpu
import jax.experimental.pallas.tpu_sc as plsc

N = 10000
E = 320000
D = 128
H = 128
O = 2
NP = 10240

NC = 2
NS = 16
NW = NC * NS
EPW = E // NW
CH = 125
NCH = EPW // CH
RPT = NP // NS

_SC_MESH = dict(core_axis_name="c", subcore_axis_name="s",
                num_cores=NC, num_subcores=NS)


def _sc_deg_body(dst_hbm, zeros_hbm, out_hbm, idx_v, hist):
    c = lax.axis_index("c")
    s = lax.axis_index("s")
    pltpu.sync_copy(zeros_hbm, hist)
    pltpu.sync_copy(dst_hbm.at[c, s], idx_v)
    one16 = jnp.ones((16,), jnp.int32)

    @pl.loop(0, EPW // 16)
    def _grp(j):
        idx = idx_v[0, pl.ds(j * 16, 16)]
        plsc.addupdate_scatter(hist.at[0], [idx], one16)

    pltpu.sync_copy(hist, out_hbm.at[c, s])


@jax.jit
def _sc_deg(dst_r2, zeros_i_hbm):
    return pl.kernel(
        _sc_deg_body,
        out_type=jax.ShapeDtypeStruct((NC, NS, 1, NP), jnp.int32),
        mesh=plsc.VectorSubcoreMesh(**_SC_MESH),
        scratch_types=[
            pltpu.VMEM((1, EPW), jnp.int32),
            pltpu.VMEM((1, NP), jnp.int32),
        ],
        compiler_params=pltpu.CompilerParams(needs_layout_passes=False),
    )(dst_r2, zeros_i_hbm)


def _sc_agg_body(hws_hbm, src_hbm, dst_hbm, zeros_hbm, out_hbm,
                 sidx_v, didx_v, rows0, rows1, acc,
                 sem0, sem1, semd0, semd1):
    c = lax.axis_index("c")
    s = lax.axis_index("s")
    pltpu.sync_copy(zeros_hbm, acc.at[pl.ds(s * RPT, RPT)])
    pltpu.sync_copy(src_hbm.at[c, s], sidx_v)
    plsc.subcore_barrier()

    rows = (rows0, rows1)
    sems = (sem0, sem1)
    semd = (semd0, semd1)
    for b in range(2):
        pltpu.async_copy(dst_hbm.at[c, s, b], didx_v.at[pl.ds(b, 1)], semd[b])
        pltpu.async_copy(hws_hbm.at[sidx_v.at[b]], rows[b], sems[b])

    @pl.loop(0, NCH, step=2)
    def _chunks(j):
        for b in range(2):
            jj = j + b
            pltpu.make_async_copy(hws_hbm.at[sidx_v.at[jj]],
                                  rows[b], sems[b]).wait()
            pltpu.make_async_copy(dst_hbm.at[c, s, jj],
                                  didx_v.at[pl.ds(b, 1)], semd[b]).wait()
            pltpu.sync_copy(rows[b], acc.at[didx_v.at[b]], add=True)
            nxt = jnp.minimum(jj + 2, NCH - 1)
            pltpu.async_copy(dst_hbm.at[c, s, nxt],
                             didx_v.at[pl.ds(b, 1)], semd[b])
            pltpu.async_copy(hws_hbm.at[sidx_v.at[nxt]], rows[b], sems[b])

    for b in range(2):
        pltpu.make_async_copy(hws_hbm.at[sidx_v.at[NCH - 1]],
                              rows[b], sems[b]).wait()
        pltpu.make_async_copy(dst_hbm.at[c, s, NCH - 1],
                              didx_v.at[pl.ds(b, 1)], semd[b]).wait()

    plsc.subcore_barrier()
    pltpu.sync_copy(acc.at[pl.ds(s * RPT, RPT)],
                    out_hbm.at[c, pl.ds(s * RPT, RPT)])


@jax.jit
def _sc_agg(hws, src_r, dst_r, zeros_hbm):
    return pl.kernel(
        _sc_agg_body,
        out_type=jax.ShapeDtypeStruct((NC, NP, H), jnp.float32),
        mesh=plsc.VectorSubcoreMesh(**_SC_MESH),
        scratch_types=[
            pltpu.VMEM((NCH, CH), jnp.int32),
            pltpu.VMEM((2, CH), jnp.int32),
            pltpu.VMEM((CH, H), jnp.float32),
            pltpu.VMEM((CH, H), jnp.float32),
            pltpu.VMEM_SHARED((NP, H), jnp.float32),
            pltpu.SemaphoreType.DMA,
            pltpu.SemaphoreType.DMA,
            pltpu.SemaphoreType.DMA,
            pltpu.SemaphoreType.DMA,
        ],
    )(hws, src_r, dst_r, zeros_hbm)


_BR = 512


def _tc_mm_body(x_ref, w1_ref, sw_ref, hw_ref, res_ref):
    xb = x_ref[...]
    hw_ref[...] = jnp.dot(xb, w1_ref[...], preferred_element_type=jnp.float32)
    res_ref[...] = jnp.dot(xb, sw_ref[...], preferred_element_type=jnp.float32)


@jax.jit
def _tc_mm(x_pad, W1, skip_W):
    grid = (NP // _BR,)
    return pl.pallas_call(
        _tc_mm_body,
        grid=grid,
        in_specs=[
            pl.BlockSpec((_BR, D), lambda i: (i, 0)),
            pl.BlockSpec((D, H), lambda i: (0, 0)),
            pl.BlockSpec((D, H), lambda i: (0, 0)),
        ],
        out_specs=[
            pl.BlockSpec((_BR, H), lambda i: (i, 0)),
            pl.BlockSpec((_BR, H), lambda i: (i, 0)),
        ],
        out_shape=[
            jax.ShapeDtypeStruct((NP, H), jnp.float32),
            jax.ShapeDtypeStruct((NP, H), jnp.float32),
        ],
    )(x_pad, W1, skip_W)


def _tc_scale_body(hw_ref, hist_ref, hws_ref, dinv_ref):
    hs = jnp.sum(hist_ref[...].astype(jnp.float32), axis=0, keepdims=True)
    d_row = lax.rsqrt(hs + 1.0)
    rows = lax.broadcasted_iota(jnp.int32, (_BR, _BR), 0)
    cols = lax.broadcasted_iota(jnp.int32, (_BR, _BR), 1)
    dm = jnp.where(rows == cols, jnp.broadcast_to(d_row, (_BR, _BR)), 0.0)
    dinv = jnp.dot(dm, jnp.ones((_BR, H), jnp.float32),
                   preferred_element_type=jnp.float32)
    hws_ref[...] = hw_ref[...] * dinv
    dinv_ref[...] = dinv


@jax.jit
def _tc_scale(hw1, hist):
    grid = (NP // _BR,)
    blk = pl.BlockSpec((_BR, H), lambda i: (i, 0))
    return pl.pallas_call(
        _tc_scale_body,
        grid=grid,
        in_specs=[blk, pl.BlockSpec((NW, _BR), lambda i: (0, i))],
        out_specs=[blk, blk],
        out_shape=[
            jax.ShapeDtypeStruct((NP, H), jnp.float32),
            jax.ShapeDtypeStruct((NP, H), jnp.float32),
        ],
    )(hw1, hist)


def _tc_mid_body(s0_ref, s1_ref, hws_ref, dinv_ref, b_ref, g_ref, be_ref,
                 w2_ref, out_ref):
    d = dinv_ref[:, :1]
    bns = g_ref[...] * lax.rsqrt(jnp.float32(1.0 + 1e-5))
    h = (s0_ref[...] + s1_ref[...] + hws_ref[...]) * d + b_ref[...]
    h = jnp.maximum(h * bns + be_ref[...], 0.0)
    out_ref[...] = jnp.dot(h, w2_ref[...],
                           preferred_element_type=jnp.float32) * d


@jax.jit
def _tc_mid(s0, s1, hws1, dinv, b1, g1, be1, W2):
    grid = (NP // _BR,)
    blk = pl.BlockSpec((_BR, H), lambda i: (i, 0))
    vec = pl.BlockSpec((1, H), lambda i: (0, 0))
    return pl.pallas_call(
        _tc_mid_body,
        grid=grid,
        in_specs=[blk, blk, blk, blk, vec, vec, vec,
                  pl.BlockSpec((H, H), lambda i: (0, 0))],
        out_specs=blk,
        out_shape=jax.ShapeDtypeStruct((NP, H), jnp.float32),
    )(s0, s1, hws1, dinv, b1, g1, be1, W2)


def _tc_fin_body(s0_ref, s1_ref, hws_ref, dinv_ref, res_ref, b_ref, g_ref,
                 be_ref, lw_ref, lb_ref, out_ref):
    d = dinv_ref[:, :1]
    bns = g_ref[...] * lax.rsqrt(jnp.float32(1.0 + 1e-5))
    h = (s0_ref[...] + s1_ref[...] + hws_ref[...]) * d + b_ref[...]
    h = jnp.maximum(h * bns + be_ref[...] + res_ref[...], 0.0)
    out_ref[...] = jnp.dot(h, lw_ref[...],
                           preferred_element_type=jnp.float32) + lb_ref[...]


@jax.jit
def _tc_fin(s0, s1, hws2, dinv, resid, b2, g2, be2, lin_W_pad, lin_b_pad):
    grid = (NP // _BR,)
    blk = pl.BlockSpec((_BR, H), lambda i: (i, 0))
    vec = pl.BlockSpec((1, H), lambda i: (0, 0))
    return pl.pallas_call(
        _tc_fin_body,
        grid=grid,
        in_specs=[blk, blk, blk, blk, blk, vec, vec, vec,
                  pl.BlockSpec((H, H), lambda i: (0, 0)), vec],
        out_specs=blk,
        out_shape=jax.ShapeDtypeStruct((NP, H), jnp.float32),
    )(s0, s1, hws2, dinv, resid, b2, g2, be2, lin_W_pad, lin_b_pad)


def kernel(x, edge_index, W1, b1, W2, b2, g1, be1, g2, be2,
           skip_W, lin_W, lin_b):
    src_r = edge_index[0].reshape(NC, NS, NCH, CH)
    dst_r2 = edge_index[1].reshape(NC, NS, 1, EPW)
    dst_r5 = edge_index[1].reshape(NC, NS, NCH, 1, CH)
    x_pad = jnp.pad(x, ((0, NP - N), (0, 0)))

    zeros_hbm = jnp.zeros((RPT, H), jnp.float32)
    zeros_i_hbm = jnp.zeros((1, NP), jnp.int32)

    degp = _sc_deg(dst_r2, zeros_i_hbm)
    hw1, resid = _tc_mm(x_pad, W1, skip_W)
    hws1, dinv = _tc_scale(hw1, degp.reshape(NW, NP))

    agg1 = _sc_agg(hws1, src_r, dst_r5, zeros_hbm)
    hws2 = _tc_mid(agg1[0], agg1[1], hws1, dinv,
                   b1.reshape(1, H), g1.reshape(1, H), be1.reshape(1, H), W2)

    agg2 = _sc_agg(hws2, src_r, dst_r5, zeros_hbm)
    lin_W_pad = jnp.pad(lin_W, ((0, 0), (0, H - O)))
    lin_b_pad = jnp.pad(lin_b, ((0, H - O),)).reshape(1, H)
    out = _tc_fin(agg2[0], agg2[1], hws2, dinv, resid,
                  b2.reshape(1, H), g2.reshape(1, H), be2.reshape(1, H),
                  lin_W_pad, lin_b_pad)
    return out[:N, :O]


# speedup vs baseline: 29.2497x; 1.0408x over previous
import functools

import jax
import jax.numpy as jnp
from jax import lax
from jax.experimental import pallas as pl
from jax.experimental.pallas import tpu as pltpu
import jax.experimental.pallas.tpu_sc as plsc

N = 10000
E = 320000
D = 128
H = 128
O = 2
NP = 10240

NC = 2
NS = 16
NW = NC * NS
EPW = E // NW
CH = 125
NCH = EPW // CH
RPT = NP // NS

_SC_MESH = dict(core_axis_name="c", subcore_axis_name="s",
                num_cores=NC, num_subcores=NS)


def _sc_deg_body(dst_hbm, zeros_hbm, out_hbm, idx_v, hist):
    c = lax.axis_index("c")
    s = lax.axis_index("s")
    pltpu.sync_copy(zeros_hbm, hist)
    pltpu.sync_copy(dst_hbm.at[c, s], idx_v)
    one16 = jnp.ones((16,), jnp.int32)

    @pl.loop(0, EPW // 16)
    def _grp(j):
        idx = idx_v[0, pl.ds(j * 16, 16)]
        plsc.addupdate_scatter(hist.at[0], [idx], one16)

    pltpu.sync_copy(hist, out_hbm.at[c, s])


@jax.jit
def _sc_deg(dst_r2, zeros_i_hbm):
    return pl.kernel(
        _sc_deg_body,
        out_type=jax.ShapeDtypeStruct((NC, NS, 1, NP), jnp.int32),
        mesh=plsc.VectorSubcoreMesh(**_SC_MESH),
        scratch_types=[
            pltpu.VMEM((1, EPW), jnp.int32),
            pltpu.VMEM((1, NP), jnp.int32),
        ],
        compiler_params=pltpu.CompilerParams(needs_layout_passes=False),
    )(dst_r2, zeros_i_hbm)


def _sc_agg_body(hws_hbm, src_hbm, dst_hbm, zeros_hbm, out_hbm,
                 sidx_v, didx_v, rows0, rows1, acc,
                 sem0, sem1, semd0, semd1):
    c = lax.axis_index("c")
    s = lax.axis_index("s")
    pltpu.sync_copy(zeros_hbm, acc.at[pl.ds(s * RPT, RPT)])
    pltpu.sync_copy(src_hbm.at[c, s], sidx_v)
    plsc.subcore_barrier()

    rows = (rows0, rows1)
    sems = (sem0, sem1)
    semd = (semd0, semd1)
    for b in range(2):
        pltpu.async_copy(dst_hbm.at[c, s, b], didx_v.at[pl.ds(b, 1)], semd[b])
        pltpu.async_copy(hws_hbm.at[sidx_v.at[b]], rows[b], sems[b])

    @pl.loop(0, NCH, step=2)
    def _chunks(j):
        for b in range(2):
            jj = j + b
            pltpu.make_async_copy(hws_hbm.at[sidx_v.at[jj]],
                                  rows[b], sems[b]).wait()
            pltpu.make_async_copy(dst_hbm.at[c, s, jj],
                                  didx_v.at[pl.ds(b, 1)], semd[b]).wait()
            pltpu.sync_copy(rows[b], acc.at[didx_v.at[b]], add=True)
            nxt = jnp.minimum(jj + 2, NCH - 1)
            pltpu.async_copy(dst_hbm.at[c, s, nxt],
                             didx_v.at[pl.ds(b, 1)], semd[b])
            pltpu.async_copy(hws_hbm.at[sidx_v.at[nxt]], rows[b], sems[b])

    for b in range(2):
        pltpu.make_async_copy(hws_hbm.at[sidx_v.at[NCH - 1]],
                              rows[b], sems[b]).wait()
        pltpu.make_async_copy(dst_hbm.at[c, s, NCH - 1],
                              didx_v.at[pl.ds(b, 1)], semd[b]).wait()

    plsc.subcore_barrier()
    pltpu.sync_copy(acc.at[pl.ds(s * RPT, RPT)],
                    out_hbm.at[c, pl.ds(s * RPT, RPT)])


@jax.jit
def _sc_agg(hws, src_r, dst_r, zeros_hbm):
    return pl.kernel(
        _sc_agg_body,
        out_type=jax.ShapeDtypeStruct((NC, NP, H), jnp.float32),
        mesh=plsc.VectorSubcoreMesh(**_SC_MESH),
        scratch_types=[
            pltpu.VMEM((NCH, CH), jnp.int32),
            pltpu.VMEM((2, CH), jnp.int32),
            pltpu.VMEM((CH, H), jnp.float32),
            pltpu.VMEM((CH, H), jnp.float32),
            pltpu.VMEM_SHARED((NP, H), jnp.float32),
            pltpu.SemaphoreType.DMA,
            pltpu.SemaphoreType.DMA,
            pltpu.SemaphoreType.DMA,
            pltpu.SemaphoreType.DMA,
        ],
    )(hws, src_r, dst_r, zeros_hbm)


_BR = 512


def _tc_prep_body(x_ref, hist_ref, w1_ref, sw_ref, hws_ref, dinv_ref, res_ref):
    hs = jnp.sum(hist_ref[...].astype(jnp.float32), axis=0, keepdims=True)
    d_row = lax.rsqrt(hs + 1.0)
    rows = lax.broadcasted_iota(jnp.int32, (_BR, _BR), 0)
    cols = lax.broadcasted_iota(jnp.int32, (_BR, _BR), 1)
    dm = jnp.where(rows == cols, jnp.broadcast_to(d_row, (_BR, _BR)), 0.0)
    dinv = jnp.dot(dm, jnp.ones((_BR, H), jnp.float32),
                   preferred_element_type=jnp.float32)
    xb = x_ref[...]
    hw = jnp.dot(xb, w1_ref[...], preferred_element_type=jnp.float32)
    hws_ref[...] = hw * dinv
    dinv_ref[...] = dinv
    res_ref[...] = jnp.dot(xb, sw_ref[...], preferred_element_type=jnp.float32)


@jax.jit
def _tc_prep(x_pad, hist, W1, skip_W):
    grid = (NP // _BR,)
    blk = pl.BlockSpec((_BR, H), lambda i: (i, 0))
    return pl.pallas_call(
        _tc_prep_body,
        grid=grid,
        in_specs=[
            pl.BlockSpec((_BR, D), lambda i: (i, 0)),
            pl.BlockSpec((NW, _BR), lambda i: (0, i)),
            pl.BlockSpec((D, H), lambda i: (0, 0)),
            pl.BlockSpec((D, H), lambda i: (0, 0)),
        ],
        out_specs=[blk, blk, blk],
        out_shape=[
            jax.ShapeDtypeStruct((NP, H), jnp.float32),
            jax.ShapeDtypeStruct((NP, H), jnp.float32),
            jax.ShapeDtypeStruct((NP, H), jnp.float32),
        ],
    )(x_pad, hist, W1, skip_W)


def _tc_mid_body(s0_ref, s1_ref, hws_ref, dinv_ref, b_ref, g_ref, be_ref,
                 w2_ref, out_ref):
    d = dinv_ref[:, :1]
    bns = g_ref[...] * lax.rsqrt(jnp.float32(1.0 + 1e-5))
    h = (s0_ref[...] + s1_ref[...] + hws_ref[...]) * d + b_ref[...]
    h = jnp.maximum(h * bns + be_ref[...], 0.0)
    out_ref[...] = jnp.dot(h, w2_ref[...],
                           preferred_element_type=jnp.float32) * d


@jax.jit
def _tc_mid(s0, s1, hws1, dinv, b1, g1, be1, W2):
    grid = (NP // _BR,)
    blk = pl.BlockSpec((_BR, H), lambda i: (i, 0))
    vec = pl.BlockSpec((1, H), lambda i: (0, 0))
    return pl.pallas_call(
        _tc_mid_body,
        grid=grid,
        in_specs=[blk, blk, blk, blk, vec, vec, vec,
                  pl.BlockSpec((H, H), lambda i: (0, 0))],
        out_specs=blk,
        out_shape=jax.ShapeDtypeStruct((NP, H), jnp.float32),
    )(s0, s1, hws1, dinv, b1, g1, be1, W2)


def _tc_fin_body(s0_ref, s1_ref, hws_ref, dinv_ref, res_ref, b_ref, g_ref,
                 be_ref, lw_ref, lb_ref, out_ref):
    d = dinv_ref[:, :1]
    bns = g_ref[...] * lax.rsqrt(jnp.float32(1.0 + 1e-5))
    h = (s0_ref[...] + s1_ref[...] + hws_ref[...]) * d + b_ref[...]
    h = jnp.maximum(h * bns + be_ref[...] + res_ref[...], 0.0)
    out_ref[...] = jnp.dot(h, lw_ref[...],
                           preferred_element_type=jnp.float32) + lb_ref[...]


@jax.jit
def _tc_fin(s0, s1, hws2, dinv, resid, b2, g2, be2, lin_W_pad, lin_b_pad):
    grid = (NP // _BR,)
    blk = pl.BlockSpec((_BR, H), lambda i: (i, 0))
    vec = pl.BlockSpec((1, H), lambda i: (0, 0))
    return pl.pallas_call(
        _tc_fin_body,
        grid=grid,
        in_specs=[blk, blk, blk, blk, blk, vec, vec, vec,
                  pl.BlockSpec((H, H), lambda i: (0, 0)), vec],
        out_specs=blk,
        out_shape=jax.ShapeDtypeStruct((NP, H), jnp.float32),
    )(s0, s1, hws2, dinv, resid, b2, g2, be2, lin_W_pad, lin_b_pad)


def kernel(x, edge_index, W1, b1, W2, b2, g1, be1, g2, be2,
           skip_W, lin_W, lin_b):
    src_r = edge_index[0].reshape(NC, NS, NCH, CH)
    dst_r2 = edge_index[1].reshape(NC, NS, 1, EPW)
    dst_r5 = edge_index[1].reshape(NC, NS, NCH, 1, CH)
    x_pad = jnp.pad(x, ((0, NP - N), (0, 0)))

    zeros_hbm = jnp.zeros((RPT, H), jnp.float32)
    zeros_i_hbm = jnp.zeros((1, NP), jnp.int32)

    degp = _sc_deg(dst_r2, zeros_i_hbm)
    hws1, dinv, resid = _tc_prep(x_pad, degp.reshape(NW, NP), W1, skip_W)

    agg1 = _sc_agg(hws1, src_r, dst_r5, zeros_hbm)
    hws2 = _tc_mid(agg1[0], agg1[1], hws1, dinv,
                   b1.reshape(1, H), g1.reshape(1, H), be1.reshape(1, H), W2)

    agg2 = _sc_agg(hws2, src_r, dst_r5, zeros_hbm)
    lin_W_pad = jnp.pad(lin_W, ((0, 0), (0, H - O)))
    lin_b_pad = jnp.pad(lin_b, ((0, H - O),)).reshape(1, H)
    out = _tc_fin(agg2[0], agg2[1], hws2, dinv, resid,
                  b2.reshape(1, H), g2.reshape(1, H), be2.reshape(1, H),
                  lin_W_pad, lin_b_pad)
    return out[:N, :O]
